# baseline (speedup 1.0000x reference)
import functools

import jax
import jax.numpy as jnp
from jax import lax
from jax.experimental import pallas as pl
from jax.experimental.pallas import tpu as pltpu
from jax.experimental.pallas import tpu_sc as plsc

_N = 8192
_BATCH = 4096
_DIM = 512
_BB = 256

_NC = 2
_NS = 16
_NW = _NC * _NS
_RPW = _BATCH // _NW
_L = 16
_CHUNKS = _N // _L
_UNROLL = 8


def _count_body(x_ref, w_ref, c_ref):
    z = jnp.dot(x_ref[...], w_ref[...], preferred_element_type=jnp.float32)
    c_ref[...] = jax.nn.sigmoid(z) * _N


def _count(x, w_count):
    return pl.pallas_call(
        _count_body,
        grid=(_BATCH // _BB,),
        in_specs=[
            pl.BlockSpec((_BB, _DIM), lambda i: (i, 0)),
            pl.BlockSpec((_DIM, 1), lambda i: (0, 0)),
        ],
        out_specs=pl.BlockSpec((_BB, 1), lambda i: (i, 0)),
        out_shape=jax.ShapeDtypeStruct((_BATCH, 1), jnp.float32),
    )(x, w_count)


_mesh = plsc.VectorSubcoreMesh(core_axis_name="c", subcore_axis_name="s")


@functools.partial(
    pl.kernel,
    mesh=_mesh,
    out_type=jax.ShapeDtypeStruct((_BATCH, _N), jnp.float32),
    scratch_types=[
        pltpu.VMEM((_RPW,), jnp.float32),
        pltpu.VMEM((2 * _N,), jnp.float32),
        pltpu.SemaphoreType.DMA,
        pltpu.SemaphoreType.DMA,
    ],
)
def _sc_gate(c_hbm, out_hbm, c_v, row_v, sem0, sem1):
    wid = lax.axis_index("s") * _NC + lax.axis_index("c")
    base = wid * _RPW
    pltpu.sync_copy(c_hbm.at[pl.ds(base, _RPW)], c_v)
    sems = (sem0, sem1)
    iota_f = lax.iota(jnp.int32, _L).astype(jnp.float32)

    def fill_row(row, buf, csplat):
        off = buf * _N

        def chunk_grp(k, idxf):
            for u in range(_UNROLL):
                kk = k * _UNROLL + u
                g16 = jnp.clip(csplat - idxf, 0.0, 1.0)
                row_v[pl.ds(off + kk * _L, _L)] = g16
                idxf = idxf + jnp.float32(_L)
            return idxf

        lax.fori_loop(0, _CHUNKS // _UNROLL, chunk_grp, iota_f, unroll=False)
        pltpu.async_copy(
            row_v.at[pl.ds(off, _N)], out_hbm.at[base + row], sems[buf])

    def wait_row(buf):
        pltpu.make_async_copy(
            row_v.at[pl.ds(buf * _N, _N)], out_hbm.at[base], sems[buf]).wait()

    def group(g, _):
        cvec = c_v[pl.ds(g * _L, _L)]
        for r in range(_L):
            buf = r % 2
            if r < 2:
                @pl.when(g > 0)
                def _wait():
                    wait_row(buf)
            else:
                wait_row(buf)
            csplat = jnp.full((_L,), cvec[r], dtype=jnp.float32)
            fill_row(g * _L + r, buf, csplat)
        return 0

    lax.fori_loop(0, _RPW // _L, group, 0, unroll=False)
    wait_row(0)
    wait_row(1)


def kernel(x, w_count):
    c = _count(x, w_count).reshape(_BATCH)
    return _sc_gate(c)


# speedup vs baseline: 2.3351x; 2.3351x over previous
import jax
import jax.numpy as jnp
from jax.experimental import pallas as pl
from jax.experimental.pallas import tpu as pltpu

_N = 8192
_BATCH = 4096
_DIM = 512
_BB = 256


def _gate_body(x_ref, w_ref, o_ref):
    z = jnp.dot(x_ref[...], w_ref[...], preferred_element_type=jnp.float32)
    c = jax.nn.sigmoid(z) * _N
    idx = jax.lax.broadcasted_iota(jnp.int32, (_BB, _N), 1).astype(jnp.float32)
    o_ref[...] = jnp.clip(c - idx, 0.0, 1.0)


def kernel(x, w_count):
    return pl.pallas_call(
        _gate_body,
        grid=(_BATCH // _BB,),
        in_specs=[
            pl.BlockSpec((_BB, _DIM), lambda i: (i, 0)),
            pl.BlockSpec((_DIM, 1), lambda i: (0, 0)),
        ],
        out_specs=pl.BlockSpec((_BB, _N), lambda i: (i, 0)),
        out_shape=jax.ShapeDtypeStruct((_BATCH, _N), jnp.float32),
    )(x, w_count)
